# baseline (speedup 1.0000x reference)
import functools

import jax
import jax.numpy as jnp
from jax import lax
from jax.experimental import pallas as pl
from jax.experimental.pallas import tpu as pltpu
from jax.experimental.pallas import tpu_sc as plsc

SEQ = 200
BATCH = 4096
DIM = 64
VOCAB = 1000000

NC = 2
NS = 16
NW = NC * NS
BPW = BATCH // NW
C1 = 128
C2 = SEQ - C1
XCH = 8
LANES = 16
NJ = DIM // LANES

BLK = 128
PITCH = 129
BWORDS = BLK * DIM
NFULL = VOCAB // BLK
TAIL0 = NFULL * BLK
TAILN = VOCAB - TAIL0
TSTEPS = NFULL // NW + 1


def _tr_body(tt_hbm, tail_hbm, pad_hbm, blk0, blk1, tb0, tb1,
             si0, si1, so0, so1):
    wid = lax.axis_index("s") * NC + lax.axis_index("c")
    lane = lax.iota(jnp.int32, LANES)
    jvecs = [lane + k * LANES for k in range(NJ)]

    def c0_of(i):
        return (wid + NW * i) * BLK

    def fire_in(i, blk, sem):
        @pl.when(c0_of(i) + BLK <= VOCAB)
        def _():
            pltpu.async_copy(tt_hbm.at[:, pl.ds(c0_of(i), BLK)],
                             blk.at[:, pl.ds(0, BLK)], sem)

    def transpose_block(blk, tb, ncols):
        def row(r, carry):
            col = jnp.full((LANES,), r, jnp.int32)
            for k in range(NJ):
                vals = plsc.load_gather(blk, [jvecs[k], col])
                tb[pl.ds(r * DIM + k * LANES, LANES)] = vals
            return carry

        lax.fori_loop(0, ncols, row, 0, unroll=8)

    fire_in(0, blk0, si0)
    fire_in(1, blk1, si1)

    def step(ii, carry):
        for ph, (blk, tb, sem_i, sem_o) in enumerate(
                ((blk0, tb0, si0, so0), (blk1, tb1, si1, so1))):
            i = 2 * ii + ph
            c0 = c0_of(i)

            @pl.when(c0 + BLK <= VOCAB)
            def _():
                pltpu.make_async_copy(tt_hbm.at[:, pl.ds(c0, BLK)],
                                      blk.at[:, pl.ds(0, BLK)],
                                      sem_i).wait()
                @pl.when(i >= 2)
                def _():
                    pltpu.make_async_copy(
                        tb, pad_hbm.at[pl.ds(c0_of(i - 2) * DIM, BWORDS)],
                        sem_o).wait()

                transpose_block(blk, tb, BLK)
                pltpu.async_copy(tb, pad_hbm.at[pl.ds(c0 * DIM, BWORDS)],
                                 sem_o)
                fire_in(i + 2, blk, sem_i)
        return carry

    lax.fori_loop(0, (TSTEPS + 1) // 2, step, 0)
    pltpu.make_async_copy(tb0, pad_hbm.at[pl.ds(0, BWORDS)], so0).wait()
    pltpu.make_async_copy(tb1, pad_hbm.at[pl.ds(0, BWORDS)], so1).wait()

    @pl.when(wid == (TAIL0 // BLK) % NW)
    def _():
        pltpu.sync_copy(tail_hbm, tb0.at[pl.ds(0, TAILN * DIM)])
        pltpu.sync_copy(tb0.at[pl.ds(0, TAILN * DIM)],
                        pad_hbm.at[pl.ds(TAIL0 * DIM, TAILN * DIM)])


def _pool_body(x_hbm, tab_hbm, out_hbm, raw_v, idx_v, rows0, rows1, out_v,
               sem0, sem1):
    wid = lax.axis_index("s") * NC + lax.axis_index("c")
    base = wid * BPW

    lane = lax.iota(jnp.int32, LANES)

    def stage_chunk(c, carry):
        s0 = c * XCH
        pltpu.sync_copy(x_hbm.at[pl.ds(s0, XCH), pl.ds(base, BPW)], raw_v)
        for sl in range(XCH):
            col = jnp.full((LANES,), s0 + sl, jnp.int32)

            def tr(k, carry2):
                vals = raw_v[sl, pl.ds(k * LANES, LANES)]
                plsc.store_scatter(
                    idx_v, [lane + k * LANES, col], vals)
                return carry2

            lax.fori_loop(0, BPW // LANES, tr, 0, unroll=4)
        return carry

    lax.fori_loop(0, SEQ // XCH, stage_chunk, 0)

    def fire(b, rows, sem):
        pltpu.async_copy(tab_hbm.at[idx_v.at[b, pl.ds(0, C1)]],
                         rows.at[pl.ds(0, C1)], sem)
        pltpu.async_copy(tab_hbm.at[idx_v.at[b, pl.ds(C1, C2)]],
                         rows.at[pl.ds(C1, C2)], sem)

    def drain(b, rows, sem):
        pltpu.make_async_copy(tab_hbm.at[idx_v.at[b, pl.ds(0, C1)]],
                              rows.at[pl.ds(0, C1)], sem).wait()
        pltpu.make_async_copy(tab_hbm.at[idx_v.at[b, pl.ds(C1, C2)]],
                              rows.at[pl.ds(C1, C2)], sem).wait()

    def reduce_rows(b, rows):
        def red(s, accs):
            return tuple(
                jnp.maximum(a, rows[s, pl.ds(j * LANES, LANES)])
                for j, a in enumerate(accs))
        init = tuple(
            jnp.full((LANES,), -jnp.inf, jnp.float32) for _ in range(NJ))
        accs = lax.fori_loop(0, SEQ, red, init, unroll=8)
        for j in range(NJ):
            out_v[b, pl.ds(j * LANES, LANES)] = accs[j]

    fire(0, rows0, sem0)
    fire(1, rows1, sem1)

    def step(i, carry):
        for ph, (rows, sem) in enumerate(((rows0, sem0), (rows1, sem1))):
            b = 2 * i + ph
            drain(b, rows, sem)
            reduce_rows(b, rows)
            nb = b + 2

            @pl.when(nb < BPW)
            def _():
                fire(nb, rows, sem)
        return carry

    lax.fori_loop(0, BPW // 2, step, 0)

    pltpu.sync_copy(out_v, out_hbm.at[pl.ds(base, BPW)])


def kernel(x, table):
    tt = table.T

    mesh = plsc.VectorSubcoreMesh(
        core_axis_name="c", subcore_axis_name="s",
        num_cores=NC, num_subcores=NS)

    transpose = functools.partial(
        pl.kernel,
        out_type=jax.ShapeDtypeStruct((VOCAB * DIM,), jnp.float32),
        mesh=mesh,
        compiler_params=pltpu.CompilerParams(
            use_tc_tiling_on_sc=True, needs_layout_passes=False),
        scratch_types=[
            pltpu.VMEM((DIM, PITCH), jnp.float32),
            pltpu.VMEM((DIM, PITCH), jnp.float32),
            pltpu.VMEM((BWORDS,), jnp.float32),
            pltpu.VMEM((BWORDS,), jnp.float32),
            pltpu.SemaphoreType.DMA,
            pltpu.SemaphoreType.DMA,
            pltpu.SemaphoreType.DMA,
            pltpu.SemaphoreType.DMA,
        ],
    )(_tr_body)

    pool = functools.partial(
        pl.kernel,
        out_type=jax.ShapeDtypeStruct((BATCH, DIM), jnp.float32),
        mesh=mesh,
        compiler_params=pltpu.CompilerParams(
            use_tc_tiling_on_sc=False, needs_layout_passes=False),
        scratch_types=[
            pltpu.VMEM((XCH, BPW), jnp.int32),
            pltpu.VMEM((BPW, SEQ), jnp.int32),
            pltpu.VMEM((SEQ, DIM), jnp.float32),
            pltpu.VMEM((SEQ, DIM), jnp.float32),
            pltpu.VMEM((BPW, DIM), jnp.float32),
            pltpu.SemaphoreType.DMA,
            pltpu.SemaphoreType.DMA,
        ],
    )(_pool_body)

    tail_flat = table[TAIL0:, :].reshape(-1)
    packed = transpose(tt, tail_flat).reshape(VOCAB, DIM)
    return pool(x, packed)


# speedup vs baseline: 1.7712x; 1.7712x over previous
import functools

import jax
import jax.numpy as jnp
from jax import lax
from jax.experimental import pallas as pl
from jax.experimental.pallas import tpu as pltpu
from jax.experimental.pallas import tpu_sc as plsc

SEQ = 200
BATCH = 4096
DIM = 64
VOCAB = 1000000

NC = 2
NS = 16
NW = NC * NS
BPW = BATCH // NW
C1 = 128
C2 = SEQ - C1
XCH = 8
LANES = 16
NJ = DIM // LANES

SLAB = 256
SWORDS = SLAB * DIM
TAIL0 = (VOCAB // SLAB) * SLAB
TAILN = VOCAB - TAIL0
NSLAB = TAIL0 // SLAB
DSTEPS = NSLAB // NW + 1


def _depad_body(tab_hbm, tail_hbm, out_hbm, in0, in1, ov0, ov1,
                si0, si1, so0, so1):
    wid = lax.axis_index("s") * NC + lax.axis_index("c")

    def c0_of(i):
        return (wid + NW * i) * SLAB

    def fire_in(i, inv, sem):
        @pl.when(c0_of(i) + SLAB <= TAIL0)
        def _():
            pltpu.async_copy(tab_hbm.at[pl.ds(c0_of(i), SLAB)], inv, sem)

    def depad_slab(inv, ov):
        def row(r, carry):
            for k in range(NJ):
                ov[pl.ds(r * DIM + k * LANES, LANES)] = (
                    inv[r, pl.ds(k * LANES, LANES)])
            return carry

        lax.fori_loop(0, SLAB, row, 0, unroll=4)

    fire_in(0, in0, si0)
    fire_in(1, in1, si1)

    def step(ii, carry):
        for ph, (inv, ov, sem_i, sem_o) in enumerate(
                ((in0, ov0, si0, so0), (in1, ov1, si1, so1))):
            i = 2 * ii + ph
            c0 = c0_of(i)

            @pl.when(c0 + SLAB <= TAIL0)
            def _():
                pltpu.make_async_copy(tab_hbm.at[pl.ds(c0, SLAB)],
                                      inv, sem_i).wait()
                @pl.when(i >= 2)
                def _():
                    pltpu.make_async_copy(
                        ov, out_hbm.at[pl.ds(c0_of(i - 2) * DIM, SWORDS)],
                        sem_o).wait()

                depad_slab(inv, ov)
                pltpu.async_copy(ov, out_hbm.at[pl.ds(c0 * DIM, SWORDS)],
                                 sem_o)
                fire_in(i + 2, inv, sem_i)
        return carry

    lax.fori_loop(0, (DSTEPS + 1) // 2, step, 0)
    pltpu.make_async_copy(ov0, out_hbm.at[pl.ds(0, SWORDS)], so0).wait()
    pltpu.make_async_copy(ov1, out_hbm.at[pl.ds(0, SWORDS)], so1).wait()

    @pl.when(wid == NSLAB % NW)
    def _():
        pltpu.sync_copy(tail_hbm, ov0.at[pl.ds(0, TAILN * DIM)])
        pltpu.sync_copy(ov0.at[pl.ds(0, TAILN * DIM)],
                        out_hbm.at[pl.ds(TAIL0 * DIM, TAILN * DIM)])


def _pool_body(x_hbm, tab_hbm, out_hbm, raw_v, idx_v, rows0, rows1, out_v,
               sem0, sem1):
    wid = lax.axis_index("s") * NC + lax.axis_index("c")
    base = wid * BPW

    lane = lax.iota(jnp.int32, LANES)

    def stage_chunk(c, carry):
        s0 = c * XCH
        pltpu.sync_copy(x_hbm.at[pl.ds(s0, XCH), pl.ds(base, BPW)], raw_v)
        for sl in range(XCH):
            col = jnp.full((LANES,), s0 + sl, jnp.int32)

            def tr(k, carry2):
                vals = raw_v[sl, pl.ds(k * LANES, LANES)]
                plsc.store_scatter(
                    idx_v, [lane + k * LANES, col], vals)
                return carry2

            lax.fori_loop(0, BPW // LANES, tr, 0, unroll=4)
        return carry

    lax.fori_loop(0, SEQ // XCH, stage_chunk, 0)

    def fire(b, rows, sem):
        pltpu.async_copy(tab_hbm.at[idx_v.at[b, pl.ds(0, C1)]],
                         rows.at[pl.ds(0, C1)], sem)
        pltpu.async_copy(tab_hbm.at[idx_v.at[b, pl.ds(C1, C2)]],
                         rows.at[pl.ds(C1, C2)], sem)

    def drain(b, rows, sem):
        pltpu.make_async_copy(tab_hbm.at[idx_v.at[b, pl.ds(0, C1)]],
                              rows.at[pl.ds(0, C1)], sem).wait()
        pltpu.make_async_copy(tab_hbm.at[idx_v.at[b, pl.ds(C1, C2)]],
                              rows.at[pl.ds(C1, C2)], sem).wait()

    def reduce_rows(b, rows):
        def red(s, accs):
            return tuple(
                jnp.maximum(a, rows[s, pl.ds(j * LANES, LANES)])
                for j, a in enumerate(accs))
        init = tuple(
            jnp.full((LANES,), -jnp.inf, jnp.float32) for _ in range(NJ))
        accs = lax.fori_loop(0, SEQ, red, init, unroll=8)
        for j in range(NJ):
            out_v[b, pl.ds(j * LANES, LANES)] = accs[j]

    fire(0, rows0, sem0)
    fire(1, rows1, sem1)

    def step(i, carry):
        for ph, (rows, sem) in enumerate(((rows0, sem0), (rows1, sem1))):
            b = 2 * i + ph
            drain(b, rows, sem)
            reduce_rows(b, rows)
            nb = b + 2

            @pl.when(nb < BPW)
            def _():
                fire(nb, rows, sem)
        return carry

    lax.fori_loop(0, BPW // 2, step, 0)

    pltpu.sync_copy(out_v, out_hbm.at[pl.ds(base, BPW)])


def kernel(x, table):
    mesh = plsc.VectorSubcoreMesh(
        core_axis_name="c", subcore_axis_name="s",
        num_cores=NC, num_subcores=NS)

    depad = functools.partial(
        pl.kernel,
        out_type=jax.ShapeDtypeStruct((VOCAB * DIM,), jnp.float32),
        mesh=mesh,
        compiler_params=pltpu.CompilerParams(
            use_tc_tiling_on_sc=True, needs_layout_passes=False),
        scratch_types=[
            pltpu.VMEM((SLAB, DIM), jnp.float32),
            pltpu.VMEM((SLAB, DIM), jnp.float32),
            pltpu.VMEM((SWORDS,), jnp.float32),
            pltpu.VMEM((SWORDS,), jnp.float32),
            pltpu.SemaphoreType.DMA,
            pltpu.SemaphoreType.DMA,
            pltpu.SemaphoreType.DMA,
            pltpu.SemaphoreType.DMA,
        ],
    )(_depad_body)

    pool = functools.partial(
        pl.kernel,
        out_type=jax.ShapeDtypeStruct((BATCH, DIM), jnp.float32),
        mesh=mesh,
        compiler_params=pltpu.CompilerParams(
            use_tc_tiling_on_sc=False, needs_layout_passes=False),
        scratch_types=[
            pltpu.VMEM((XCH, BPW), jnp.int32),
            pltpu.VMEM((BPW, SEQ), jnp.int32),
            pltpu.VMEM((SEQ, DIM), jnp.float32),
            pltpu.VMEM((SEQ, DIM), jnp.float32),
            pltpu.VMEM((BPW, DIM), jnp.float32),
            pltpu.SemaphoreType.DMA,
            pltpu.SemaphoreType.DMA,
        ],
    )(_pool_body)

    tail_flat = table[TAIL0:, :].reshape(-1)
    packed = depad(table, tail_flat).reshape(VOCAB, DIM)
    return pool(x, packed)


# speedup vs baseline: 2.2653x; 1.2790x over previous
import functools

import jax
import jax.numpy as jnp
from jax import lax
from jax.experimental import pallas as pl
from jax.experimental.pallas import tpu as pltpu
from jax.experimental.pallas import tpu_sc as plsc

SEQ = 200
BATCH = 4096
DIM = 64
VOCAB = 1000000

NC = 2
NS = 16
NW = NC * NS
BPW = BATCH // NW
C1 = 128
C2 = SEQ - C1
LANES = 16
NJ = DIM // LANES


def _pool_body(x_hbm, table_hbm, out_hbm, raw_v, idx_v, rows0, rows1, out_v,
               sem0, sem1):
    wid = lax.axis_index("s") * NC + lax.axis_index("c")
    base = wid * BPW

    pltpu.sync_copy(x_hbm.at[:, pl.ds(base, BPW)], raw_v)

    lane = lax.iota(jnp.int32, LANES)

    def transpose_step(s, carry):
        col = jnp.full((LANES,), s, jnp.int32)
        for k in range(BPW // LANES):
            vals = raw_v[s, pl.ds(k * LANES, LANES)]
            plsc.store_scatter(idx_v, [lane + (k * LANES), col], vals)
        return carry

    lax.fori_loop(0, SEQ, transpose_step, 0, unroll=2)

    def fire(b, rows, sem):
        pltpu.async_copy(table_hbm.at[idx_v.at[b, pl.ds(0, C1)]],
                         rows.at[pl.ds(0, C1)], sem)
        pltpu.async_copy(table_hbm.at[idx_v.at[b, pl.ds(C1, C2)]],
                         rows.at[pl.ds(C1, C2)], sem)

    def drain(b, rows, sem):
        pltpu.make_async_copy(table_hbm.at[idx_v.at[b, pl.ds(0, C1)]],
                              rows.at[pl.ds(0, C1)], sem).wait()
        pltpu.make_async_copy(table_hbm.at[idx_v.at[b, pl.ds(C1, C2)]],
                              rows.at[pl.ds(C1, C2)], sem).wait()

    def reduce_rows(b, rows):
        def red(s, accs):
            return tuple(
                jnp.maximum(a, rows[s, pl.ds(j * LANES, LANES)])
                for j, a in enumerate(accs))
        init = tuple(
            jnp.full((LANES,), -jnp.inf, jnp.float32) for _ in range(NJ))
        accs = lax.fori_loop(0, SEQ, red, init, unroll=8)
        for j in range(NJ):
            out_v[b, pl.ds(j * LANES, LANES)] = accs[j]

    fire(0, rows0, sem0)
    fire(1, rows1, sem1)

    def step(i, carry):
        for ph, (rows, sem) in enumerate(((rows0, sem0), (rows1, sem1))):
            b = 2 * i + ph
            drain(b, rows, sem)
            reduce_rows(b, rows)
            nb = b + 2

            @pl.when(nb < BPW)
            def _():
                fire(nb, rows, sem)
        return carry

    lax.fori_loop(0, BPW // 2, step, 0)

    pltpu.sync_copy(out_v, out_hbm.at[pl.ds(base, BPW)])


def kernel(x, table):
    mesh = plsc.VectorSubcoreMesh(
        core_axis_name="c", subcore_axis_name="s",
        num_cores=NC, num_subcores=NS)

    pool = functools.partial(
        pl.kernel,
        out_type=jax.ShapeDtypeStruct((BATCH, DIM), jnp.float32),
        mesh=mesh,
        compiler_params=pltpu.CompilerParams(
            use_tc_tiling_on_sc=False, needs_layout_passes=False),
        scratch_types=[
            pltpu.VMEM((SEQ, BPW), jnp.int32),
            pltpu.VMEM((BPW, SEQ), jnp.int32),
            pltpu.VMEM((SEQ, DIM), jnp.float32),
            pltpu.VMEM((SEQ, DIM), jnp.float32),
            pltpu.VMEM((BPW, DIM), jnp.float32),
            pltpu.SemaphoreType.DMA,
            pltpu.SemaphoreType.DMA,
        ],
    )(_pool_body)

    return pool(x, table)


# speedup vs baseline: 2.2671x; 1.0008x over previous
import functools

import jax
import jax.numpy as jnp
from jax import lax
from jax.experimental import pallas as pl
from jax.experimental.pallas import tpu as pltpu
from jax.experimental.pallas import tpu_sc as plsc

SEQ = 200
BATCH = 4096
DIM = 64
VOCAB = 1000000

NC = 2
NS = 16
NW = NC * NS
BPW = BATCH // NW
C1 = 128
C2 = SEQ - C1
LANES = 16
NJ = DIM // LANES


def _pool_body(xt_hbm, table_hbm, out_hbm, idx_v, rows0, rows1, out_v,
               sem0, sem1):
    wid = lax.axis_index("s") * NC + lax.axis_index("c")
    base = wid * BPW

    pltpu.sync_copy(xt_hbm.at[pl.ds(base, BPW)], idx_v)

    def fire(b, rows, sem):
        pltpu.async_copy(table_hbm.at[idx_v.at[b, pl.ds(0, C1)]],
                         rows.at[pl.ds(0, C1)], sem)
        pltpu.async_copy(table_hbm.at[idx_v.at[b, pl.ds(C1, C2)]],
                         rows.at[pl.ds(C1, C2)], sem)

    def drain(b, rows, sem):
        pltpu.make_async_copy(table_hbm.at[idx_v.at[b, pl.ds(0, C1)]],
                              rows.at[pl.ds(0, C1)], sem).wait()
        pltpu.make_async_copy(table_hbm.at[idx_v.at[b, pl.ds(C1, C2)]],
                              rows.at[pl.ds(C1, C2)], sem).wait()

    def reduce_rows(b, rows):
        def red(s, accs):
            return tuple(
                jnp.maximum(a, rows[s, pl.ds(j * LANES, LANES)])
                for j, a in enumerate(accs))
        init = tuple(
            jnp.full((LANES,), -jnp.inf, jnp.float32) for _ in range(NJ))
        accs = lax.fori_loop(0, SEQ, red, init, unroll=8)
        for j in range(NJ):
            out_v[b, pl.ds(j * LANES, LANES)] = accs[j]

    fire(0, rows0, sem0)
    fire(1, rows1, sem1)

    def step(i, carry):
        for ph, (rows, sem) in enumerate(((rows0, sem0), (rows1, sem1))):
            b = 2 * i + ph
            drain(b, rows, sem)
            reduce_rows(b, rows)
            nb = b + 2

            @pl.when(nb < BPW)
            def _():
                fire(nb, rows, sem)
        return carry

    lax.fori_loop(0, BPW // 2, step, 0)

    pltpu.sync_copy(out_v, out_hbm.at[pl.ds(base, BPW)])


def kernel(x, table):
    mesh = plsc.VectorSubcoreMesh(
        core_axis_name="c", subcore_axis_name="s",
        num_cores=NC, num_subcores=NS)

    pool = functools.partial(
        pl.kernel,
        out_type=jax.ShapeDtypeStruct((BATCH, DIM), jnp.float32),
        mesh=mesh,
        compiler_params=pltpu.CompilerParams(
            use_tc_tiling_on_sc=False, needs_layout_passes=False),
        scratch_types=[
            pltpu.VMEM((BPW, SEQ), jnp.int32),
            pltpu.VMEM((SEQ, DIM), jnp.float32),
            pltpu.VMEM((SEQ, DIM), jnp.float32),
            pltpu.VMEM((BPW, DIM), jnp.float32),
            pltpu.SemaphoreType.DMA,
            pltpu.SemaphoreType.DMA,
        ],
    )(_pool_body)

    return pool(x.T, table)
